# baseline (device time: 57852 ns/iter reference)
import jax
import jax.numpy as jnp
import numpy as np
from jax import lax
from jax.experimental import pallas as pl
from jax.experimental.pallas import tpu as pltpu

K = 32
ROWS = 1024
BLK = 256
INT_MIN = int(np.iinfo(np.int32).min)


def _sortable(b):
    return jnp.where(b >= 0, b, INT_MIN - b)


def kernel(x):
    n_loc = x.shape[1]

    def body(x_hbm, out_ref, xv, cand, load_sem, ysems, gsend, grecv):
        mx = lax.axis_index("x")
        my = lax.axis_index("y")
        mz = lax.axis_index("z")
        rb = mx * 2 + mz
        row0 = rb * BLK

        peers = [
            (mx, 1 - my, mz),
            (1 - mx, my, mz),
            (mx, my, 1 - mz),
            (1 - mx, my, 1 - mz),
        ]

        cp = pltpu.make_async_copy(
            x_hbm.at[pl.ds(row0, BLK), :], xv, load_sem
        )
        cp.start()

        bar = pltpu.get_barrier_semaphore()
        for p in peers:
            pl.semaphore_signal(
                bar, inc=1, device_id=p, device_id_type=pl.DeviceIdType.MESH
            )

        cp.wait()

        xb = xv[:, :].astype(jnp.bfloat16).astype(jnp.float32)
        b = lax.bitcast_convert_type(xb, jnp.int32)
        col = lax.broadcasted_iota(jnp.int32, (BLK, n_loc), 1)
        k = _sortable(b) + col + my * n_loc

        for i in range(K):
            m = jnp.max(k, axis=1, keepdims=True)
            cand[0, :, pl.ds(i, 1)] = m
            cand[2, :, pl.ds(K - 1 - i, 1)] = m
            if i < K - 1:
                k = jnp.where(k == m, INT_MIN, k)

        pl.semaphore_wait(bar, 4)

        rdy = pltpu.make_async_remote_copy(
            src_ref=cand.at[2],
            dst_ref=cand.at[1],
            send_sem=ysems.at[0],
            recv_sem=ysems.at[1],
            device_id=(mx, 1 - my, mz),
            device_id_type=pl.DeviceIdType.MESH,
        )
        rdy.start()
        rdy.wait()

        t = jnp.maximum(cand[0], cand[1])
        lane = lax.broadcasted_iota(jnp.int32, (BLK, K), 1)
        for d in (16, 8, 4, 2, 1):
            up = pltpu.roll(t, K - d, axis=1)
            dn = pltpu.roll(t, d, axis=1)
            t = jnp.where(
                (lane & d) == 0, jnp.maximum(t, up), jnp.minimum(t, dn)
            )
        fin = t

        s2 = fin & jnp.int32(-65536)
        vals = lax.bitcast_convert_type(_sortable(s2), jnp.float32)
        out_ref[pl.ds(row0, BLK), :] = vals

        gpeers = [
            (1 - mx, my, mz),
            (mx, my, 1 - mz),
            (1 - mx, my, 1 - mz),
        ]
        sends = []
        for slot, p in enumerate(gpeers):
            rd = pltpu.make_async_remote_copy(
                src_ref=out_ref.at[pl.ds(row0, BLK), :],
                dst_ref=out_ref.at[pl.ds(row0, BLK), :],
                send_sem=gsend.at[slot],
                recv_sem=grecv.at[slot],
                device_id=p,
                device_id_type=pl.DeviceIdType.MESH,
            )
            rd.start()
            sends.append(rd)
        for slot, p in enumerate(gpeers):
            px, _, pz = p
            pr0 = (px * 2 + pz) * BLK
            rc = pltpu.make_async_remote_copy(
                src_ref=out_ref.at[pl.ds(pr0, BLK), :],
                dst_ref=out_ref.at[pl.ds(pr0, BLK), :],
                send_sem=gsend.at[slot],
                recv_sem=grecv.at[slot],
                device_id=p,
                device_id_type=pl.DeviceIdType.MESH,
            )
            rc.wait_recv()
        for rd in sends:
            rd.wait_send()

    return pl.pallas_call(
        body,
        out_shape=jax.ShapeDtypeStruct((ROWS, K), jnp.float32),
        in_specs=[pl.BlockSpec(memory_space=pl.ANY)],
        out_specs=pl.BlockSpec(memory_space=pltpu.VMEM),
        scratch_shapes=[
            pltpu.VMEM((BLK, n_loc), jnp.float32),
            pltpu.VMEM((3, BLK, K), jnp.int32),
            pltpu.SemaphoreType.DMA,
            pltpu.SemaphoreType.DMA((2,)),
            pltpu.SemaphoreType.DMA((3,)),
            pltpu.SemaphoreType.DMA((3,)),
        ],
        compiler_params=pltpu.CompilerParams(collective_id=0),
    )(x)
